# baseline (speedup 1.0000x reference)
import functools

import jax
import jax.numpy as jnp
from jax import lax
from jax.experimental import pallas as pl
from jax.experimental.pallas import tpu as pltpu
from jax.experimental.pallas import tpu_sc as plsc

N = 2048
D = 2048
F = 4096
NE = 8

TA = 256
NTA = N // TA

TT = 512
GMAX = 11
NDISP = GMAX * TT

FC = 256
NFC = F // FC

CH = 32
NW = 32
TPW = N // NW


def _router_body(x_ref, gwt_ref, logits_ref, pos_ref, cnt_out_ref,
                 cnt_scr, base_scr, lg_scr):
    p = pl.program_id(0)
    g = pl.program_id(1)

    @pl.when((p == 0) & (g == 0))
    def _init():
        cnt_scr[...] = jnp.zeros_like(cnt_scr)

    row0 = pl.multiple_of(g * TA, TA)

    @pl.when(p == 0)
    def _mklogits():
        lg_scr[pl.ds(row0, TA), :] = jnp.dot(
            x_ref[...], gwt_ref[...], preferred_element_type=jnp.float32)

    logits = lg_scr[pl.ds(row0, TA), :]
    lane = lax.broadcasted_iota(jnp.int32, (TA, 128), 1)
    masked = jnp.where(lane < NE, logits, jnp.float32(-1e30))
    mx = jnp.max(masked, axis=1, keepdims=True)
    top1 = jnp.min(jnp.where(masked == mx, lane, 128), axis=1,
                   keepdims=True)
    oh = (lane == top1).astype(jnp.float32)

    logits_ref[...] = logits

    @pl.when((p == 1) & (g == 0))
    def _mkbase():
        cnt = cnt_scr[...]
        ci = cnt.astype(jnp.int32)
        padded = (((ci + (TT - 1)) // TT) * TT).astype(jnp.float32)
        r2 = lax.broadcasted_iota(jnp.int32, (128, 128), 0)
        c2 = lax.broadcasted_iota(jnp.int32, (128, 128), 1)
        tri = (r2 < c2).astype(jnp.float32)
        base_scr[...] = jnp.dot(padded, tri,
                                preferred_element_type=jnp.float32)
        cnt_scr[...] = jnp.zeros_like(cnt_scr)

    @pl.when(p == 1)
    def _assign():
        ri = lax.broadcasted_iota(jnp.int32, (TA, TA), 0)
        ci = lax.broadcasted_iota(jnp.int32, (TA, TA), 1)
        tril = (ci < ri).astype(jnp.float32)
        ranks = jnp.dot(tril, oh, preferred_element_type=jnp.float32)
        rank = jnp.sum(ranks * oh, axis=1, keepdims=True)
        run = jnp.sum(cnt_scr[...] * oh, axis=1, keepdims=True)
        bas = jnp.sum(base_scr[...] * oh, axis=1, keepdims=True)
        pos = bas + run + rank
        pos_ref[...] = jnp.broadcast_to(pos, (TA, 128)).astype(jnp.int32)

    cnt_scr[...] = cnt_scr[...] + jnp.sum(oh, axis=0, keepdims=True)
    cnt_out_ref[...] = jnp.broadcast_to(cnt_scr[...], (TA, 128))


def _run_router(x, gwt_pad):
    return pl.pallas_call(
        _router_body,
        grid=(2, NTA),
        in_specs=[
            pl.BlockSpec((TA, D), lambda p, g: ((1 - p) * g, 0)),
            pl.BlockSpec((D, 128), lambda p, g: (0, 0)),
        ],
        out_specs=[
            pl.BlockSpec((TA, 128), lambda p, g: (g, 0)),
            pl.BlockSpec((TA, 128), lambda p, g: (g, 0)),
            pl.BlockSpec((TA, 128), lambda p, g: (g, 0)),
        ],
        out_shape=[
            jax.ShapeDtypeStruct((N, 128), jnp.float32),
            jax.ShapeDtypeStruct((N, 128), jnp.int32),
            jax.ShapeDtypeStruct((N, 128), jnp.float32),
        ],
        scratch_shapes=[
            pltpu.VMEM((1, 128), jnp.float32),
            pltpu.VMEM((1, 128), jnp.float32),
            pltpu.VMEM((N, 128), jnp.float32),
        ],
    )(x, gwt_pad)


@functools.lru_cache(maxsize=None)
def _sc_kernels():
    mesh = plsc.VectorSubcoreMesh(core_axis_name="c", subcore_axis_name="s")

    @functools.partial(
        pl.kernel,
        mesh=mesh,
        out_type=jax.ShapeDtypeStruct((NDISP, D), jnp.float32),
        scratch_types=[
            pltpu.VMEM((CH,), jnp.int32),
            pltpu.VMEM((CH, D), jnp.float32),
            pltpu.SemaphoreType.DMA,
        ],
    )
    def sc_dispatch(x_hbm, cpos_hbm, xdisp_hbm, idx_v, rows_v, sem):
        wid = lax.axis_index("s") * 2 + lax.axis_index("c")
        base = wid * TPW
        for j in range(TPW // CH):
            b = base + j * CH
            pltpu.sync_copy(cpos_hbm.at[pl.ds(b, CH)], idx_v)
            pltpu.sync_copy(x_hbm.at[pl.ds(b, CH)], rows_v)
            pltpu.async_copy(rows_v, xdisp_hbm.at[idx_v], sem).wait()

    @functools.partial(
        pl.kernel,
        mesh=mesh,
        out_type=jax.ShapeDtypeStruct((N, D), jnp.float32),
        scratch_types=[
            pltpu.VMEM((CH,), jnp.int32),
            pltpu.VMEM((CH, D), jnp.float32),
            pltpu.SemaphoreType.DMA,
        ],
    )
    def sc_combine(ydisp_hbm, cpos_hbm, out_hbm, idx_v, rows_v, sem):
        wid = lax.axis_index("s") * 2 + lax.axis_index("c")
        base = wid * TPW
        for j in range(TPW // CH):
            b = base + j * CH
            pltpu.sync_copy(cpos_hbm.at[pl.ds(b, CH)], idx_v)
            pltpu.async_copy(ydisp_hbm.at[idx_v], rows_v, sem).wait()
            pltpu.sync_copy(rows_v, out_hbm.at[pl.ds(b, CH)])

    return sc_dispatch, sc_combine


FH = FC // 2
D2 = D // 2
TH = TT // 2


def _ffn_body(te_ref, nr_ref, xb_ref, xd_ref,
              wga_ref, wgb_ref, wua_ref, wub_ref, wda_ref, wdb_ref, out_ref):
    g = pl.program_id(0)
    f = pl.program_id(1)

    dn = (((1,), (1,)), ((), ()))

    def rows(lo):
        xt = xd_ref[lo:lo + TH, :]

        def half(wg_ref, wu_ref):
            gg = lax.dot_general(xt, wg_ref[0, 0], dn,
                                 preferred_element_type=jnp.float32)
            uu = lax.dot_general(xt, wu_ref[0, 0], dn,
                                 preferred_element_type=jnp.float32)
            return gg * jax.nn.sigmoid(gg) * uu

        hh = jnp.concatenate(
            [half(wga_ref, wua_ref), half(wgb_ref, wub_ref)], axis=1)
        pa = lax.dot_general(hh, wda_ref[0], dn,
                             preferred_element_type=jnp.float32)
        pb = lax.dot_general(hh, wdb_ref[0], dn,
                             preferred_element_type=jnp.float32)

        @pl.when(f == 0)
        def _():
            out_ref[lo:lo + TH, :D2] = pa
            out_ref[lo:lo + TH, D2:] = pb

        @pl.when(f > 0)
        def _():
            out_ref[lo:lo + TH, :D2] = out_ref[lo:lo + TH, :D2] + pa
            out_ref[lo:lo + TH, D2:] = out_ref[lo:lo + TH, D2:] + pb

    @pl.when(nr_ref[g] > 0)
    def _():
        rows(0)

    @pl.when(nr_ref[g] > TH)
    def _():
        rows(TH)


def _run_ffn(tile_expert, tile_nrows, tile_xblk, xdisp, wg, wu, wd):
    wgr = wg.reshape(NE, 2 * NFC, FH, D)
    wur = wu.reshape(NE, 2 * NFC, FH, D)
    grid_spec = pltpu.PrefetchScalarGridSpec(
        num_scalar_prefetch=3,
        grid=(GMAX, NFC),
        in_specs=[
            pl.BlockSpec((TT, D), lambda g, f, te, nr, xb: (xb[g], 0)),
            pl.BlockSpec((1, 1, FH, D),
                         lambda g, f, te, nr, xb: (te[g], 2 * f, 0, 0)),
            pl.BlockSpec((1, 1, FH, D),
                         lambda g, f, te, nr, xb: (te[g], 2 * f + 1, 0, 0)),
            pl.BlockSpec((1, 1, FH, D),
                         lambda g, f, te, nr, xb: (te[g], 2 * f, 0, 0)),
            pl.BlockSpec((1, 1, FH, D),
                         lambda g, f, te, nr, xb: (te[g], 2 * f + 1, 0, 0)),
            pl.BlockSpec((1, D2, FC),
                         lambda g, f, te, nr, xb: (te[g], 0, f)),
            pl.BlockSpec((1, D2, FC),
                         lambda g, f, te, nr, xb: (te[g], 1, f)),
        ],
        out_specs=pl.BlockSpec((TT, D), lambda g, f, te, nr, xb: (g, 0)),
    )
    return pl.pallas_call(
        _ffn_body,
        grid_spec=grid_spec,
        out_shape=jax.ShapeDtypeStruct((NDISP, D), jnp.float32),
        compiler_params=pltpu.CompilerParams(
            dimension_semantics=("parallel", "arbitrary")),
    )(tile_expert, tile_nrows, tile_xblk, xdisp, wgr, wgr, wur, wur, wd, wd)


@jax.jit
def kernel(hidden_states, gate_w, gate_proj_w, up_proj_w, down_proj_w):
    b, s, d = hidden_states.shape
    x = hidden_states.reshape(N, D)

    gwt_pad = jnp.zeros((D, 128), jnp.float32).at[:, :NE].set(gate_w.T)
    logits_pad, pos_out, cnt_out = _run_router(x, gwt_pad)
    router_logits = logits_pad[:, :NE]
    cpos = pos_out[:, 0]
    counts = cnt_out[-1, :NE].astype(jnp.int32)

    tiles_per_e = (counts + (TT - 1)) // TT
    tile_expert = jnp.repeat(jnp.arange(NE, dtype=jnp.int32), tiles_per_e,
                             total_repeat_length=GMAX)
    base_tile = jnp.cumsum(tiles_per_e) - tiles_per_e
    slot = jnp.arange(GMAX, dtype=jnp.int32) - base_tile[tile_expert]
    tile_nrows = jnp.clip(counts[tile_expert] - slot * TT, 0, TT)
    nvalid = jnp.sum(tiles_per_e).astype(jnp.int32)
    tile_xblk = jnp.minimum(jnp.arange(GMAX, dtype=jnp.int32), nvalid - 1)

    sc_dispatch, sc_combine = _sc_kernels()
    xdisp = sc_dispatch(x, cpos)
    ydisp = _run_ffn(tile_expert, tile_nrows, tile_xblk, xdisp,
                     gate_proj_w, up_proj_w, down_proj_w)
    final = sc_combine(ydisp, cpos)

    return (final.reshape(b, s, d), router_logits)


# speedup vs baseline: 1.3176x; 1.3176x over previous
import functools

import jax
import jax.numpy as jnp
from jax import lax
from jax.experimental import pallas as pl
from jax.experimental.pallas import tpu as pltpu
from jax.experimental.pallas import tpu_sc as plsc

N = 2048
D = 2048
F = 4096
NE = 8

TA = 256
NTA = N // TA

TT = 512
GMAX = 11
NDISP = GMAX * TT

FC = 1024
NFC = F // FC

CH = 32
NW = 32
TPW = N // NW


def _router_body(x_ref, gwt_ref, logits_ref, pos_ref, cnt_out_ref,
                 cnt_scr, base_scr, lg_scr):
    p = pl.program_id(0)
    g = pl.program_id(1)

    @pl.when((p == 0) & (g == 0))
    def _init():
        cnt_scr[...] = jnp.zeros_like(cnt_scr)

    row0 = pl.multiple_of(g * TA, TA)

    @pl.when(p == 0)
    def _mklogits():
        lg_scr[pl.ds(row0, TA), :] = jnp.dot(
            x_ref[...], gwt_ref[...], preferred_element_type=jnp.float32)

    logits = lg_scr[pl.ds(row0, TA), :]
    lane = lax.broadcasted_iota(jnp.int32, (TA, 128), 1)
    masked = jnp.where(lane < NE, logits, jnp.float32(-1e30))
    mx = jnp.max(masked, axis=1, keepdims=True)
    top1 = jnp.min(jnp.where(masked == mx, lane, 128), axis=1,
                   keepdims=True)
    oh = (lane == top1).astype(jnp.float32)

    logits_ref[...] = logits

    @pl.when((p == 1) & (g == 0))
    def _mkbase():
        cnt = cnt_scr[...]
        ci = cnt.astype(jnp.int32)
        padded = (((ci + (TT - 1)) // TT) * TT).astype(jnp.float32)
        r2 = lax.broadcasted_iota(jnp.int32, (128, 128), 0)
        c2 = lax.broadcasted_iota(jnp.int32, (128, 128), 1)
        tri = (r2 < c2).astype(jnp.float32)
        base_scr[...] = jnp.dot(padded, tri,
                                preferred_element_type=jnp.float32)
        cnt_scr[...] = jnp.zeros_like(cnt_scr)

    @pl.when(p == 1)
    def _assign():
        ri = lax.broadcasted_iota(jnp.int32, (TA, TA), 0)
        ci = lax.broadcasted_iota(jnp.int32, (TA, TA), 1)
        tril = (ci < ri).astype(jnp.float32)
        ranks = jnp.dot(tril, oh, preferred_element_type=jnp.float32)
        rank = jnp.sum(ranks * oh, axis=1, keepdims=True)
        run = jnp.sum(cnt_scr[...] * oh, axis=1, keepdims=True)
        bas = jnp.sum(base_scr[...] * oh, axis=1, keepdims=True)
        pos = bas + run + rank
        pos_ref[...] = jnp.broadcast_to(pos, (TA, 128)).astype(jnp.int32)

    cnt_scr[...] = cnt_scr[...] + jnp.sum(oh, axis=0, keepdims=True)
    cnt_out_ref[...] = jnp.broadcast_to(cnt_scr[...], (TA, 128))


def _run_router(x, gwt_pad):
    return pl.pallas_call(
        _router_body,
        grid=(2, NTA),
        in_specs=[
            pl.BlockSpec((TA, D), lambda p, g: ((1 - p) * g, 0)),
            pl.BlockSpec((D, 128), lambda p, g: (0, 0)),
        ],
        out_specs=[
            pl.BlockSpec((TA, 128), lambda p, g: (g, 0)),
            pl.BlockSpec((TA, 128), lambda p, g: (g, 0)),
            pl.BlockSpec((TA, 128), lambda p, g: (g, 0)),
        ],
        out_shape=[
            jax.ShapeDtypeStruct((N, 128), jnp.float32),
            jax.ShapeDtypeStruct((N, 128), jnp.int32),
            jax.ShapeDtypeStruct((N, 128), jnp.float32),
        ],
        scratch_shapes=[
            pltpu.VMEM((1, 128), jnp.float32),
            pltpu.VMEM((1, 128), jnp.float32),
            pltpu.VMEM((N, 128), jnp.float32),
        ],
    )(x, gwt_pad)


@functools.lru_cache(maxsize=None)
def _sc_kernels():
    mesh = plsc.VectorSubcoreMesh(core_axis_name="c", subcore_axis_name="s")

    @functools.partial(
        pl.kernel,
        mesh=mesh,
        out_type=jax.ShapeDtypeStruct((NDISP, D), jnp.float32),
        scratch_types=[
            pltpu.VMEM((CH,), jnp.int32),
            pltpu.VMEM((CH, D), jnp.float32),
            pltpu.SemaphoreType.DMA,
        ],
    )
    def sc_dispatch(x_hbm, cpos_hbm, xdisp_hbm, idx_v, rows_v, sem):
        wid = lax.axis_index("s") * 2 + lax.axis_index("c")
        base = wid * TPW
        for j in range(TPW // CH):
            b = base + j * CH
            pltpu.sync_copy(cpos_hbm.at[pl.ds(b, CH)], idx_v)
            pltpu.sync_copy(x_hbm.at[pl.ds(b, CH)], rows_v)
            pltpu.async_copy(rows_v, xdisp_hbm.at[idx_v], sem).wait()

    @functools.partial(
        pl.kernel,
        mesh=mesh,
        out_type=jax.ShapeDtypeStruct((N, D), jnp.float32),
        scratch_types=[
            pltpu.VMEM((CH,), jnp.int32),
            pltpu.VMEM((CH, D), jnp.float32),
            pltpu.SemaphoreType.DMA,
        ],
    )
    def sc_combine(ydisp_hbm, cpos_hbm, out_hbm, idx_v, rows_v, sem):
        wid = lax.axis_index("s") * 2 + lax.axis_index("c")
        base = wid * TPW
        for j in range(TPW // CH):
            b = base + j * CH
            pltpu.sync_copy(cpos_hbm.at[pl.ds(b, CH)], idx_v)
            pltpu.async_copy(ydisp_hbm.at[idx_v], rows_v, sem).wait()
            pltpu.sync_copy(rows_v, out_hbm.at[pl.ds(b, CH)])

    return sc_dispatch, sc_combine


FH = FC // 2
D2 = D // 2
TH = TT // 2


def _ffn_body(te_ref, nr_ref, xb_ref, xd_ref,
              wga_ref, wgb_ref, wua_ref, wub_ref, wda_ref, wdb_ref, out_ref):
    g = pl.program_id(0)
    f = pl.program_id(1)

    dn = (((1,), (1,)), ((), ()))

    def rows(lo):
        xt = xd_ref[lo:lo + TH, :]

        def half(wg_ref, wu_ref):
            gg = lax.dot_general(xt, wg_ref[0, 0], dn,
                                 preferred_element_type=jnp.float32)
            uu = lax.dot_general(xt, wu_ref[0, 0], dn,
                                 preferred_element_type=jnp.float32)
            return gg * jax.nn.sigmoid(gg) * uu

        hh = jnp.concatenate(
            [half(wga_ref, wua_ref), half(wgb_ref, wub_ref)], axis=1)
        pa = lax.dot_general(hh, wda_ref[0], dn,
                             preferred_element_type=jnp.float32)
        pb = lax.dot_general(hh, wdb_ref[0], dn,
                             preferred_element_type=jnp.float32)

        @pl.when(f == 0)
        def _():
            out_ref[lo:lo + TH, :D2] = pa
            out_ref[lo:lo + TH, D2:] = pb

        @pl.when(f > 0)
        def _():
            out_ref[lo:lo + TH, :D2] = out_ref[lo:lo + TH, :D2] + pa
            out_ref[lo:lo + TH, D2:] = out_ref[lo:lo + TH, D2:] + pb

    @pl.when(nr_ref[g] > 0)
    def _():
        rows(0)

    @pl.when(nr_ref[g] > TH)
    def _():
        rows(TH)


def _run_ffn(tile_expert, tile_nrows, tile_xblk, xdisp, wg, wu, wd):
    wgr = wg.reshape(NE, 2 * NFC, FH, D)
    wur = wu.reshape(NE, 2 * NFC, FH, D)
    grid_spec = pltpu.PrefetchScalarGridSpec(
        num_scalar_prefetch=3,
        grid=(GMAX, NFC),
        in_specs=[
            pl.BlockSpec((TT, D), lambda g, f, te, nr, xb: (xb[g], 0),
                         pipeline_mode=pl.Buffered(buffer_count=1)),
            pl.BlockSpec((1, 1, FH, D),
                         lambda g, f, te, nr, xb: (te[g], 2 * f, 0, 0)),
            pl.BlockSpec((1, 1, FH, D),
                         lambda g, f, te, nr, xb: (te[g], 2 * f + 1, 0, 0)),
            pl.BlockSpec((1, 1, FH, D),
                         lambda g, f, te, nr, xb: (te[g], 2 * f, 0, 0)),
            pl.BlockSpec((1, 1, FH, D),
                         lambda g, f, te, nr, xb: (te[g], 2 * f + 1, 0, 0)),
            pl.BlockSpec((1, D2, FC),
                         lambda g, f, te, nr, xb: (te[g], 0, f)),
            pl.BlockSpec((1, D2, FC),
                         lambda g, f, te, nr, xb: (te[g], 1, f)),
        ],
        out_specs=pl.BlockSpec((TT, D), lambda g, f, te, nr, xb: (g, 0)),
    )
    return pl.pallas_call(
        _ffn_body,
        grid_spec=grid_spec,
        out_shape=jax.ShapeDtypeStruct((NDISP, D), jnp.float32),
        compiler_params=pltpu.CompilerParams(
            dimension_semantics=("parallel", "arbitrary"),
            vmem_limit_bytes=63 * 1024 * 1024),
    )(tile_expert, tile_nrows, tile_xblk, xdisp, wgr, wgr, wur, wur, wd, wd)


@jax.jit
def kernel(hidden_states, gate_w, gate_proj_w, up_proj_w, down_proj_w):
    b, s, d = hidden_states.shape
    x = hidden_states.reshape(N, D)

    gwt_pad = jnp.zeros((D, 128), jnp.float32).at[:, :NE].set(gate_w.T)
    logits_pad, pos_out, cnt_out = _run_router(x, gwt_pad)
    router_logits = logits_pad[:, :NE]
    cpos = pos_out[:, 0]
    counts = cnt_out[-1, :NE].astype(jnp.int32)

    tiles_per_e = (counts + (TT - 1)) // TT
    tile_expert = jnp.repeat(jnp.arange(NE, dtype=jnp.int32), tiles_per_e,
                             total_repeat_length=GMAX)
    base_tile = jnp.cumsum(tiles_per_e) - tiles_per_e
    slot = jnp.arange(GMAX, dtype=jnp.int32) - base_tile[tile_expert]
    tile_nrows = jnp.clip(counts[tile_expert] - slot * TT, 0, TT)
    nvalid = jnp.sum(tiles_per_e).astype(jnp.int32)
    tile_xblk = jnp.minimum(jnp.arange(GMAX, dtype=jnp.int32), nvalid - 1)

    sc_dispatch, sc_combine = _sc_kernels()
    xdisp = sc_dispatch(x, cpos)
    ydisp = _run_ffn(tile_expert, tile_nrows, tile_xblk, xdisp,
                     gate_proj_w, up_proj_w, down_proj_w)
    final = sc_combine(ydisp, cpos)

    return (final.reshape(b, s, d), router_logits)


# speedup vs baseline: 1.3597x; 1.0319x over previous
import functools

import jax
import jax.numpy as jnp
from jax import lax
from jax.experimental import pallas as pl
from jax.experimental.pallas import tpu as pltpu
from jax.experimental.pallas import tpu_sc as plsc

N = 2048
D = 2048
F = 4096
NE = 8

TA = 256
NTA = N // TA

TT = 512
GMAX = 11
NDISP = GMAX * TT

FC = 512
NFC = F // FC

CH = 32
NW = 32
TPW = N // NW


def _router_body(x_ref, gw_ref, logits_ref, pos_ref, cnt_out_ref,
                 cnt_scr, base_scr, lg_scr):
    p = pl.program_id(0)
    g = pl.program_id(1)

    @pl.when((p == 0) & (g == 0))
    def _init():
        cnt_scr[...] = jnp.zeros_like(cnt_scr)

    row0 = pl.multiple_of(g * TA, TA)

    @pl.when(p == 0)
    def _mklogits():
        lg_scr[pl.ds(row0, TA), :] = lax.dot_general(
            x_ref[...], gw_ref[...], (((1,), (1,)), ((), ())),
            preferred_element_type=jnp.float32)

    logits = lg_scr[pl.ds(row0, TA), :]
    lane = lax.broadcasted_iota(jnp.int32, (TA, NE), 1)
    mx = jnp.max(logits, axis=1, keepdims=True)
    top1 = jnp.min(jnp.where(logits == mx, lane, NE), axis=1,
                   keepdims=True)
    oh = (lane == top1).astype(jnp.float32)

    logits_ref[...] = logits

    @pl.when((p == 1) & (g == 0))
    def _mkbase():
        cnt = cnt_scr[...]
        ci = cnt.astype(jnp.int32)
        padded = (((ci + (TT - 1)) // TT) * TT).astype(jnp.float32)
        r2 = lax.broadcasted_iota(jnp.int32, (NE, NE), 0)
        c2 = lax.broadcasted_iota(jnp.int32, (NE, NE), 1)
        tri = (r2 < c2).astype(jnp.float32)
        base_scr[...] = jnp.dot(padded, tri,
                                preferred_element_type=jnp.float32)
        cnt_scr[...] = jnp.zeros_like(cnt_scr)

    @pl.when(p == 1)
    def _assign():
        ri = lax.broadcasted_iota(jnp.int32, (TA, TA), 0)
        ci = lax.broadcasted_iota(jnp.int32, (TA, TA), 1)
        tril = (ci < ri).astype(jnp.float32)
        ranks = jnp.dot(tril, oh, preferred_element_type=jnp.float32)
        rank = jnp.sum(ranks * oh, axis=1, keepdims=True)
        run = jnp.sum(cnt_scr[...] * oh, axis=1, keepdims=True)
        bas = jnp.sum(base_scr[...] * oh, axis=1, keepdims=True)
        pos = bas + run + rank
        pos_ref[...] = jnp.broadcast_to(pos, (TA, 128)).astype(jnp.int32)

    cnt_scr[...] = cnt_scr[...] + jnp.sum(oh, axis=0, keepdims=True)
    cnt_out_ref[...] = jnp.broadcast_to(cnt_scr[...], (TA, NE))


def _run_router(x, gate_w):
    return pl.pallas_call(
        _router_body,
        grid=(2, NTA),
        in_specs=[
            pl.BlockSpec((TA, D), lambda p, g: ((1 - p) * g, 0)),
            pl.BlockSpec((NE, D), lambda p, g: (0, 0)),
        ],
        out_specs=[
            pl.BlockSpec((TA, NE), lambda p, g: (g, 0)),
            pl.BlockSpec((TA, 128), lambda p, g: (g, 0)),
            pl.BlockSpec((TA, NE), lambda p, g: (g, 0)),
        ],
        out_shape=[
            jax.ShapeDtypeStruct((N, NE), jnp.float32),
            jax.ShapeDtypeStruct((N, 128), jnp.int32),
            jax.ShapeDtypeStruct((N, NE), jnp.float32),
        ],
        scratch_shapes=[
            pltpu.VMEM((1, NE), jnp.float32),
            pltpu.VMEM((1, NE), jnp.float32),
            pltpu.VMEM((N, NE), jnp.float32),
        ],
    )(x, gate_w)


@functools.lru_cache(maxsize=None)
def _sc_kernels():
    mesh = plsc.VectorSubcoreMesh(core_axis_name="c", subcore_axis_name="s")

    @functools.partial(
        pl.kernel,
        mesh=mesh,
        out_type=jax.ShapeDtypeStruct((NDISP, D), jnp.float32),
        scratch_types=[
            pltpu.VMEM((CH,), jnp.int32),
            pltpu.VMEM((CH, D), jnp.float32),
            pltpu.SemaphoreType.DMA,
        ],
    )
    def sc_dispatch(x_hbm, cpos_hbm, xdisp_hbm, idx_v, rows_v, sem):
        wid = lax.axis_index("s") * 2 + lax.axis_index("c")
        base = wid * TPW
        for j in range(TPW // CH):
            b = base + j * CH
            pltpu.sync_copy(cpos_hbm.at[pl.ds(b, CH)], idx_v)
            pltpu.sync_copy(x_hbm.at[pl.ds(b, CH)], rows_v)
            pltpu.async_copy(rows_v, xdisp_hbm.at[idx_v], sem).wait()

    @functools.partial(
        pl.kernel,
        mesh=mesh,
        out_type=jax.ShapeDtypeStruct((N, D), jnp.float32),
        scratch_types=[
            pltpu.VMEM((CH,), jnp.int32),
            pltpu.VMEM((CH, D), jnp.float32),
            pltpu.SemaphoreType.DMA,
        ],
    )
    def sc_combine(ydisp_hbm, cpos_hbm, out_hbm, idx_v, rows_v, sem):
        wid = lax.axis_index("s") * 2 + lax.axis_index("c")
        base = wid * TPW
        for j in range(TPW // CH):
            b = base + j * CH
            pltpu.sync_copy(cpos_hbm.at[pl.ds(b, CH)], idx_v)
            pltpu.async_copy(ydisp_hbm.at[idx_v], rows_v, sem).wait()
            pltpu.sync_copy(rows_v, out_hbm.at[pl.ds(b, CH)])

    return sc_dispatch, sc_combine


FH = FC // 2
D2 = D // 2
TH = TT // 2


def _ffn_body(te_ref, nr_ref, xb_ref, xd_ref,
              wga_ref, wgb_ref, wua_ref, wub_ref, wda_ref, wdb_ref, out_ref):
    g = pl.program_id(0)
    f = pl.program_id(1)

    dn = (((1,), (1,)), ((), ()))

    def rows(lo):
        xt = xd_ref[lo:lo + TH, :]

        def half(wg_ref, wu_ref):
            gg = lax.dot_general(xt, wg_ref[0, 0], dn,
                                 preferred_element_type=jnp.float32)
            uu = lax.dot_general(xt, wu_ref[0, 0], dn,
                                 preferred_element_type=jnp.float32)
            return gg * jax.nn.sigmoid(gg) * uu

        hh = jnp.concatenate(
            [half(wga_ref, wua_ref), half(wgb_ref, wub_ref)], axis=1)
        pa = lax.dot_general(hh, wda_ref[0], dn,
                             preferred_element_type=jnp.float32)
        pb = lax.dot_general(hh, wdb_ref[0], dn,
                             preferred_element_type=jnp.float32)

        @pl.when(f == 0)
        def _():
            out_ref[lo:lo + TH, :D2] = pa
            out_ref[lo:lo + TH, D2:] = pb

        @pl.when(f > 0)
        def _():
            out_ref[lo:lo + TH, :D2] = out_ref[lo:lo + TH, :D2] + pa
            out_ref[lo:lo + TH, D2:] = out_ref[lo:lo + TH, D2:] + pb

    @pl.when(nr_ref[g] > 0)
    def _():
        rows(0)

    @pl.when(nr_ref[g] > TH)
    def _():
        rows(TH)


def _run_ffn(tile_expert, tile_nrows, tile_xblk, xdisp, wg, wu, wd):
    wgr = wg.reshape(NE, 2 * NFC, FH, D)
    wur = wu.reshape(NE, 2 * NFC, FH, D)
    grid_spec = pltpu.PrefetchScalarGridSpec(
        num_scalar_prefetch=3,
        grid=(GMAX, NFC),
        in_specs=[
            pl.BlockSpec((TT, D), lambda g, f, te, nr, xb: (xb[g], 0)),
            pl.BlockSpec((1, 1, FH, D),
                         lambda g, f, te, nr, xb: (te[g], 2 * f, 0, 0)),
            pl.BlockSpec((1, 1, FH, D),
                         lambda g, f, te, nr, xb: (te[g], 2 * f + 1, 0, 0)),
            pl.BlockSpec((1, 1, FH, D),
                         lambda g, f, te, nr, xb: (te[g], 2 * f, 0, 0)),
            pl.BlockSpec((1, 1, FH, D),
                         lambda g, f, te, nr, xb: (te[g], 2 * f + 1, 0, 0)),
            pl.BlockSpec((1, D2, FC),
                         lambda g, f, te, nr, xb: (te[g], 0, f)),
            pl.BlockSpec((1, D2, FC),
                         lambda g, f, te, nr, xb: (te[g], 1, f)),
        ],
        out_specs=pl.BlockSpec((TT, D), lambda g, f, te, nr, xb: (g, 0)),
    )
    return pl.pallas_call(
        _ffn_body,
        grid_spec=grid_spec,
        out_shape=jax.ShapeDtypeStruct((NDISP, D), jnp.float32),
        compiler_params=pltpu.CompilerParams(
            dimension_semantics=("parallel", "arbitrary"),
            vmem_limit_bytes=63 * 1024 * 1024),
    )(tile_expert, tile_nrows, tile_xblk, xdisp, wgr, wgr, wur, wur, wd, wd)


@jax.jit
def kernel(hidden_states, gate_w, gate_proj_w, up_proj_w, down_proj_w):
    b, s, d = hidden_states.shape
    x = hidden_states.reshape(N, D)

    router_logits, pos_out, cnt_out = _run_router(x, gate_w)
    cpos = pos_out[:, 0]
    counts = cnt_out[-1].astype(jnp.int32)

    tiles_per_e = (counts + (TT - 1)) // TT
    tile_expert = jnp.repeat(jnp.arange(NE, dtype=jnp.int32), tiles_per_e,
                             total_repeat_length=GMAX)
    base_tile = jnp.cumsum(tiles_per_e) - tiles_per_e
    slot = jnp.arange(GMAX, dtype=jnp.int32) - base_tile[tile_expert]
    tile_nrows = jnp.clip(counts[tile_expert] - slot * TT, 0, TT)
    nvalid = jnp.sum(tiles_per_e).astype(jnp.int32)
    tile_xblk = jnp.minimum(jnp.arange(GMAX, dtype=jnp.int32), nvalid - 1)

    sc_dispatch, sc_combine = _sc_kernels()
    xdisp = sc_dispatch(x, cpos)
    ydisp = _run_ffn(tile_expert, tile_nrows, tile_xblk, xdisp,
                     gate_proj_w, up_proj_w, down_proj_w)
    final = sc_combine(ydisp, cpos)

    return (final.reshape(b, s, d), router_logits)
